# baseline (speedup 1.0000x reference)
import jax
import jax.numpy as jnp
from jax.experimental import pallas as pl


def _sqdist(src, dst):
    return (jnp.sum(src ** 2, -1)[:, :, None] + jnp.sum(dst ** 2, -1)[:, None, :]
            - 2.0 * jnp.einsum("bnc,bmc->bnm", src, dst))


def _gather(points, idx):
    b = points.shape[0]
    batch = jnp.arange(b).reshape((b,) + (1,) * (idx.ndim - 1))
    return points[batch, idx]


def _fps(xyz, npoint):
    b, n, _ = xyz.shape

    def body(i, state):
        centroids, distance, farthest = state
        centroids = centroids.at[:, i].set(farthest)
        centroid = jnp.take_along_axis(xyz, farthest[:, None, None], axis=1)
        dist = jnp.sum((xyz - centroid) ** 2, -1)
        distance = jnp.minimum(distance, dist)
        farthest = jnp.argmax(distance, -1).astype(jnp.int32)
        return centroids, distance, farthest

    init = (jnp.zeros((b, npoint), jnp.int32),
            jnp.full((b, n), 1e10, jnp.float32),
            jnp.zeros((b,), jnp.int32))
    centroids, _, _ = jax.lax.fori_loop(0, npoint, body, init)
    return centroids


def _ball_query(radius, k, xyz, new_xyz):
    b, n, _ = xyz.shape
    s = new_xyz.shape[1]
    sqr = _sqdist(new_xyz, xyz)
    mask = sqr <= radius * radius
    pos = jnp.cumsum(mask.astype(jnp.int32), axis=-1)
    dest = jnp.where(mask & (pos <= k), pos - 1, k).reshape(b * s, n)
    rows = jnp.arange(b * s, dtype=jnp.int32)[:, None]
    cols = jnp.broadcast_to(jnp.arange(n, dtype=jnp.int32)[None, :], (b * s, n))
    buf = jnp.full((b * s, k + 1), n, jnp.int32).at[rows, dest].set(cols, mode="drop")
    gi = buf[:, :k].reshape(b, s, k)
    first = jnp.broadcast_to(gi[:, :, :1], gi.shape)
    return jnp.where(gi == n, first, gi)


def _pick_bo(cout, cin):
    bo = cout
    while bo * cin * 4 > 4 * 1024 * 1024 and bo % 2 == 0:
        bo //= 2
    return bo


def _mm_stats(x2d, w, bvec, bm):
    cin, m = x2d.shape
    cout = w.shape[0]
    nm = m // bm
    bo = _pick_bo(cout, cin)
    no = cout // bo

    def kern(x_ref, w_ref, b_ref, y_ref, s1_ref, s2_ref):
        wq = w_ref[...].astype(jnp.bfloat16)
        xq = x_ref[...].astype(jnp.bfloat16)
        y = jnp.dot(wq, xq,
                    preferred_element_type=jnp.float32) + b_ref[...]
        y_ref[...] = y
        ps1 = jnp.sum(y, axis=1, keepdims=True)
        ps2 = jnp.sum(y * y, axis=1, keepdims=True)

        @pl.when(pl.program_id(1) == 0)
        def _():
            s1_ref[...] = ps1
            s2_ref[...] = ps2

        @pl.when(pl.program_id(1) != 0)
        def _():
            s1_ref[...] += ps1
            s2_ref[...] += ps2

    y, s1, s2 = pl.pallas_call(
        kern,
        grid=(no, nm),
        in_specs=[
            pl.BlockSpec((cin, bm), lambda i, j: (0, j)),
            pl.BlockSpec((bo, cin), lambda i, j: (i, 0)),
            pl.BlockSpec((bo, 1), lambda i, j: (i, 0)),
        ],
        out_specs=[
            pl.BlockSpec((bo, bm), lambda i, j: (i, j)),
            pl.BlockSpec((bo, 1), lambda i, j: (i, 0)),
            pl.BlockSpec((bo, 1), lambda i, j: (i, 0)),
        ],
        out_shape=[
            jax.ShapeDtypeStruct((cout, m), jnp.float32),
            jax.ShapeDtypeStruct((cout, 1), jnp.float32),
            jax.ShapeDtypeStruct((cout, 1), jnp.float32),
        ],
    )(x2d, w, bvec.reshape(cout, 1))
    mean = s1[:, 0] / m
    var = s2[:, 0] / m - mean * mean
    return y, mean, var


def _scale_shift(layer, mean, var):
    scale = layer["gamma"] / jnp.sqrt(var + 1e-5)
    shift = layer["beta"] - mean * scale
    return scale.reshape(-1, 1), shift.reshape(-1, 1)


def _norm_relu(y, scale, shift, bm):
    cout, m = y.shape
    nm = m // bm

    def kern(y_ref, sc_ref, sh_ref, o_ref):
        o_ref[...] = jnp.maximum(y_ref[...] * sc_ref[...] + sh_ref[...], 0.0)

    return pl.pallas_call(
        kern,
        grid=(nm,),
        in_specs=[
            pl.BlockSpec((cout, bm), lambda j: (0, j)),
            pl.BlockSpec((cout, 1), lambda j: (0, 0)),
            pl.BlockSpec((cout, 1), lambda j: (0, 0)),
        ],
        out_specs=pl.BlockSpec((cout, bm), lambda j: (0, j)),
        out_shape=jax.ShapeDtypeStruct((cout, m), jnp.float32),
    )(y, scale, shift)


def _norm_relu_segpool(y, scale, shift, b, k, s):
    cout, m = y.shape

    def kern(y_ref, sc_ref, sh_ref, o_ref):
        z = jnp.maximum(y_ref[...] * sc_ref[...] + sh_ref[...], 0.0)

        @pl.when(pl.program_id(1) == 0)
        def _():
            o_ref[...] = z

        @pl.when(pl.program_id(1) != 0)
        def _():
            o_ref[...] = jnp.maximum(o_ref[...], z)

    return pl.pallas_call(
        kern,
        grid=(b, k),
        in_specs=[
            pl.BlockSpec((cout, s), lambda bi, ki: (0, bi * k + ki)),
            pl.BlockSpec((cout, 1), lambda bi, ki: (0, 0)),
            pl.BlockSpec((cout, 1), lambda bi, ki: (0, 0)),
        ],
        out_specs=pl.BlockSpec((cout, s), lambda bi, ki: (0, bi)),
        out_shape=jax.ShapeDtypeStruct((cout, b * s), jnp.float32),
    )(y, scale, shift)


def _chain(x2d, layers, pool=None):
    cur = x2d
    m = x2d.shape[1]
    bm = min(2048, m)
    for li, layer in enumerate(layers):
        y, mean, var = _mm_stats(cur, layer["W"], layer["b"], bm)
        sc, sh = _scale_shift(layer, mean, var)
        if li < len(layers) - 1 or pool is None:
            cur = _norm_relu(y, sc, sh, bm)
        else:
            cur = _norm_relu_segpool(y, sc, sh, pool[1], pool[2], pool[3])
    return cur


def _sa(xyz, points, npoint, radii, ks, branches):
    b = xyz.shape[0]
    fps_idx = _fps(xyz, npoint)
    new_xyz = _gather(xyz, fps_idx)
    s = npoint
    outs = []
    for i, radius in enumerate(radii):
        k = ks[i]
        gi = _ball_query(radius, k, xyz, new_xyz)
        gx = _gather(xyz, gi) - new_xyz[:, :, None, :]
        grouped = gx if points is None else jnp.concatenate(
            [_gather(points, gi), gx], axis=-1)
        c = grouped.shape[-1]
        x2d = jnp.transpose(grouped, (3, 0, 2, 1)).reshape(c, b * k * s)
        if s == 1:
            out2d = _chain(x2d, branches[i], None)
            outs.append(jnp.max(out2d.reshape(-1, b, k), axis=2))
        else:
            outs.append(_chain(x2d, branches[i], ("seg", b, k, s)))
    np2d = jnp.concatenate(outs, axis=0)
    new_points = np2d.reshape(-1, b, s).transpose(1, 2, 0)
    return new_xyz, new_points


def _fp(xyz1, xyz2, points1, points2, layers):
    b, n, _ = xyz1.shape
    s = xyz2.shape[1]
    if s == 1:
        interp = jnp.broadcast_to(points2, (b, n, points2.shape[2]))
    else:
        d = _sqdist(xyz1, xyz2)
        idx = jnp.argsort(d, axis=-1)[:, :, :3]
        d3 = jnp.take_along_axis(d, idx, axis=-1)
        recip = 1.0 / (d3 + 1e-8)
        w = recip / jnp.sum(recip, axis=2, keepdims=True)
        interp = jnp.sum(_gather(points2, idx) * w[..., None], axis=2)
    newp = interp if points1 is None else jnp.concatenate([points1, interp], -1)
    c = newp.shape[-1]
    x2d = newp.reshape(b * n, c).T
    out = _chain(x2d, layers)
    return out.reshape(-1, b, n).transpose(1, 2, 0)


@jax.jit
def kernel(xyz, params):
    b = xyz.shape[0]
    l1_xyz, l1_p = _sa(xyz, None, 512, [0.05, 0.1, 0.2], [16, 32, 64],
                       params["sa1"])
    l2_xyz, l2_p = _sa(l1_xyz, l1_p, 128, [0.1, 0.2, 0.4], [16, 32, 64],
                       params["sa2"])
    l3_xyz, l3_p = _sa(l2_xyz, l2_p, 1, [0.4, 0.8, 1.2], [32, 64, 128],
                       params["sa3"])
    l2_p = _fp(l2_xyz, l3_xyz, l2_p, l3_p, params["fp3"])
    l1_p = _fp(l1_xyz, l2_xyz, l1_p, l2_p, params["fp2"])
    l0_p = _fp(xyz, l1_xyz, None, l1_p, params["fp1"])

    c3 = l3_p.shape[-1]
    xg = l3_p.reshape(b, c3).T
    g_global = _chain(xg, [params["proj_global"]]).T

    n = l0_p.shape[1]
    xl = l0_p.reshape(b * n, l0_p.shape[-1]).T
    gl2d = _chain(xl, [params["proj_local"]])
    g_local = jnp.max(gl2d.reshape(-1, b, n), axis=2).T

    return jnp.concatenate([g_global, g_local], axis=1)
